# Initial kernel scaffold; baseline (speedup 1.0000x reference)
import jax
import jax.numpy as jnp
from jax.experimental import pallas as pl


def kernel(x, edge_index, W_l, W_r, b_l):
    raise NotImplementedError("write your pallas kernel here")


# speedup vs baseline: 2.8609x; 2.8609x over previous
import functools

import jax
import jax.numpy as jnp
from jax import lax
from jax.experimental import pallas as pl
from jax.experimental.pallas import tpu as pltpu
from jax.experimental.pallas import tpu_sc as plsc

_N = 10000
_D = 128
_NC = 2
_NS = 16
_NW = _NC * _NS
_NPAD = 10112
_RPT = _NPAD // _NS
_CHUNK = 128
_CHUNKS_PER_W = 80
_EPW = _CHUNK * _CHUNKS_PER_W
_EPAD = _EPW * _NW
_CNTW = 16
_WCHUNKS = [_CHUNK] * (_RPT // _CHUNK) + ([_RPT % _CHUNK] if _RPT % _CHUNK else [])
_RBLK = 1000


def _matmul_body(x_ref, w_ref, y_ref):
    y_ref[...] = jnp.dot(x_ref[...], w_ref[...], preferred_element_type=jnp.float32)


def _matmul(x, w):
    return pl.pallas_call(
        _matmul_body,
        grid=(_N // _RBLK,),
        in_specs=[
            pl.BlockSpec((_RBLK, _D), lambda i: (i, 0)),
            pl.BlockSpec((_D, _D), lambda i: (0, 0)),
        ],
        out_specs=pl.BlockSpec((_RBLK, _D), lambda i: (i, 0)),
        out_shape=jax.ShapeDtypeStruct((_N, _D), jnp.float32),
    )(x, w)


def _combine_body(a0_ref, a1_ref, c0_ref, c1_ref, x_ref, w_ref, b_ref, o_ref):
    cnt = jnp.maximum(c0_ref[:, 0:1] + c1_ref[:, 0:1], 1.0)
    mean = (a0_ref[...] + a1_ref[...]) / cnt
    z = mean + jnp.dot(x_ref[...], w_ref[...], preferred_element_type=jnp.float32) + b_ref[...]
    o_ref[...] = jnp.maximum(z, 0.0)


def _combine(a0, a1, c0, c1, x, w, b):
    return pl.pallas_call(
        _combine_body,
        grid=(_N // _RBLK,),
        in_specs=[
            pl.BlockSpec((_RBLK, _D), lambda i: (i, 0)),
            pl.BlockSpec((_RBLK, _D), lambda i: (i, 0)),
            pl.BlockSpec((_RBLK, _D), lambda i: (i, 0)),
            pl.BlockSpec((_RBLK, _D), lambda i: (i, 0)),
            pl.BlockSpec((_RBLK, _D), lambda i: (i, 0)),
            pl.BlockSpec((_D, _D), lambda i: (0, 0)),
            pl.BlockSpec((1, _D), lambda i: (0, 0)),
        ],
        out_specs=pl.BlockSpec((_RBLK, _D), lambda i: (i, 0)),
        out_shape=jax.ShapeDtypeStruct((_N, _D), jnp.float32),
    )(a0, a1, c0, c1, x, w, b)


def _sc_acc_body(y_hbm, src_hbm, dst_hbm, zrow_hbm,
                 acc_hbm,
                 src_v, dst_v, rows_v, acc_sh, gsem):
    c = lax.axis_index("c")
    s = lax.axis_index("s")
    wid = c * _NS + s
    rbase = s * _RPT

    pltpu.sync_copy(zrow_hbm, rows_v)
    for k, n in enumerate(_WCHUNKS):
        pltpu.sync_copy(rows_v.at[pl.ds(0, n)],
                        acc_sh.at[pl.ds(rbase + k * _CHUNK, n)])
    plsc.subcore_barrier()

    def body(j, carry):
        ebase = wid * _EPW + j * _CHUNK
        pltpu.sync_copy(src_hbm.at[pl.ds(ebase, _CHUNK)], src_v)
        pltpu.sync_copy(dst_hbm.at[pl.ds(ebase, _CHUNK)], dst_v)
        pltpu.async_copy(y_hbm.at[src_v], rows_v, gsem).wait()
        pltpu.sync_copy(rows_v, acc_sh.at[dst_v], add=True)
        return carry

    lax.fori_loop(0, _CHUNKS_PER_W, body, 0)
    plsc.subcore_barrier()

    obase = c * _NPAD + rbase
    for k, n in enumerate(_WCHUNKS):
        pltpu.sync_copy(acc_sh.at[pl.ds(rbase + k * _CHUNK, n)],
                        rows_v.at[pl.ds(0, n)])
        pltpu.sync_copy(rows_v.at[pl.ds(0, n)],
                        acc_hbm.at[pl.ds(obase + k * _CHUNK, n)])


def _sc_cnt_body(dst_hbm, ones_hbm, zrow_hbm,
                 cnt_hbm,
                 dst_v, ones_v, stage_v, cnt_sh):
    c = lax.axis_index("c")
    s = lax.axis_index("s")
    wid = c * _NS + s
    rbase = s * _RPT

    pltpu.sync_copy(zrow_hbm, stage_v)
    for k, n in enumerate(_WCHUNKS):
        pltpu.sync_copy(stage_v.at[pl.ds(0, n)],
                        cnt_sh.at[pl.ds(rbase + k * _CHUNK, n)])
    pltpu.sync_copy(ones_hbm, ones_v)
    plsc.subcore_barrier()

    def body(j, carry):
        ebase = wid * _EPW + j * _CHUNK
        pltpu.sync_copy(dst_hbm.at[pl.ds(ebase, _CHUNK)], dst_v)
        pltpu.sync_copy(ones_v, cnt_sh.at[dst_v], add=True)
        return carry

    lax.fori_loop(0, _CHUNKS_PER_W, body, 0)
    plsc.subcore_barrier()

    obase = c * _NPAD + rbase
    for k, n in enumerate(_WCHUNKS):
        pltpu.sync_copy(cnt_sh.at[pl.ds(rbase + k * _CHUNK, n)],
                        stage_v.at[pl.ds(0, n)])
        pltpu.sync_copy(stage_v.at[pl.ds(0, n)],
                        cnt_hbm.at[pl.ds(obase + k * _CHUNK, n)])


@functools.cache
def _sc_acc_call():
    return functools.partial(
        pl.kernel,
        mesh=plsc.VectorSubcoreMesh(core_axis_name="c", subcore_axis_name="s"),
        out_type=jax.ShapeDtypeStruct((_NC * _NPAD, _D), jnp.float32),
        scratch_types=[
            pltpu.VMEM((_CHUNK,), jnp.int32),
            pltpu.VMEM((_CHUNK,), jnp.int32),
            pltpu.VMEM((_CHUNK, _D), jnp.float32),
            pltpu.VMEM_SHARED((_NPAD, _D), jnp.float32),
            pltpu.SemaphoreType.DMA,
        ],
    )(_sc_acc_body)


@functools.cache
def _sc_cnt_call():
    return functools.partial(
        pl.kernel,
        mesh=plsc.VectorSubcoreMesh(core_axis_name="c", subcore_axis_name="s"),
        out_type=jax.ShapeDtypeStruct((_NC * _NPAD, _D), jnp.float32),
        scratch_types=[
            pltpu.VMEM((_CHUNK,), jnp.int32),
            pltpu.VMEM((_CHUNK, _D), jnp.float32),
            pltpu.VMEM((_CHUNK, _D), jnp.float32),
            pltpu.VMEM_SHARED((_NPAD, _D), jnp.float32),
        ],
    )(_sc_cnt_body)


def kernel(x, edge_index, W_l, W_r, b_l):
    src = edge_index[0].astype(jnp.int32)
    dst = edge_index[1].astype(jnp.int32)
    e = src.shape[0]
    pad = _EPAD - e
    src = jnp.concatenate([src, jnp.zeros((pad,), jnp.int32)])
    dst = jnp.concatenate([dst, jnp.full((pad,), _NPAD - 1, jnp.int32)])

    y = _matmul(x, W_l)

    ones = jnp.ones((_CHUNK, _D), jnp.float32)
    zrow = jnp.zeros((_CHUNK, _D), jnp.float32)
    acc = _sc_acc_call()(y, src, dst, zrow)
    cnt = _sc_cnt_call()(dst, ones, zrow)

    a0 = acc[:_N]
    a1 = acc[_NPAD:_NPAD + _N]
    c0 = cnt[:_N]
    c1 = cnt[_NPAD:_NPAD + _N]
    return _combine(a0, a1, c0, c1, x, W_r, b_l.reshape(1, _D))


# speedup vs baseline: 3.7011x; 1.2937x over previous
import functools

import jax
import jax.numpy as jnp
from jax import lax
from jax.experimental import pallas as pl
from jax.experimental.pallas import tpu as pltpu
from jax.experimental.pallas import tpu_sc as plsc

_N = 10000
_D = 128
_NC = 2
_NS = 16
_NW = _NC * _NS
_NPAD = 10112
_RPT = _NPAD // _NS
_CHUNK = 128
_CHUNKS_PER_W = 80
_EPW = _CHUNK * _CHUNKS_PER_W
_EPAD = _EPW * _NW
_CNTW = 16
_WCHUNKS = [_CHUNK] * (_RPT // _CHUNK) + ([_RPT % _CHUNK] if _RPT % _CHUNK else [])
_RBLK = 1000


def _matmul_body(x_ref, w_ref, y_ref):
    y_ref[...] = jnp.dot(x_ref[...], w_ref[...], preferred_element_type=jnp.float32)


def _matmul(x, w):
    return pl.pallas_call(
        _matmul_body,
        grid=(_N // _RBLK,),
        in_specs=[
            pl.BlockSpec((_RBLK, _D), lambda i: (i, 0)),
            pl.BlockSpec((_D, _D), lambda i: (0, 0)),
        ],
        out_specs=pl.BlockSpec((_RBLK, _D), lambda i: (i, 0)),
        out_shape=jax.ShapeDtypeStruct((_N, _D), jnp.float32),
    )(x, w)


def _combine_body(a0_ref, a1_ref, c0_ref, c1_ref, x_ref, w_ref, b_ref, o_ref):
    cnt = jnp.maximum(c0_ref[:, 0:1] + c1_ref[:, 0:1], 1.0)
    mean = (a0_ref[...] + a1_ref[...]) / cnt
    z = mean + jnp.dot(x_ref[...], w_ref[...], preferred_element_type=jnp.float32) + b_ref[...]
    o_ref[...] = jnp.maximum(z, 0.0)


def _combine(a0, a1, c0, c1, x, w, b):
    return pl.pallas_call(
        _combine_body,
        grid=(_N // _RBLK,),
        in_specs=[
            pl.BlockSpec((_RBLK, _D), lambda i: (i, 0)),
            pl.BlockSpec((_RBLK, _D), lambda i: (i, 0)),
            pl.BlockSpec((_RBLK, _D), lambda i: (i, 0)),
            pl.BlockSpec((_RBLK, _D), lambda i: (i, 0)),
            pl.BlockSpec((_RBLK, _D), lambda i: (i, 0)),
            pl.BlockSpec((_D, _D), lambda i: (0, 0)),
            pl.BlockSpec((1, _D), lambda i: (0, 0)),
        ],
        out_specs=pl.BlockSpec((_RBLK, _D), lambda i: (i, 0)),
        out_shape=jax.ShapeDtypeStruct((_N, _D), jnp.float32),
    )(a0, a1, c0, c1, x, w, b)


def _sc_acc_body(y_hbm, ei_hbm, zrow_hbm,
                 acc_hbm,
                 idx_a, idx_b, rows_a, rows_b, acc_sh, sem_a, sem_b):
    c = lax.axis_index("c")
    s = lax.axis_index("s")
    wid = c * _NS + s
    rbase = s * _RPT
    cbase = wid * _CHUNKS_PER_W

    pltpu.sync_copy(zrow_hbm, rows_a)
    for k, n in enumerate(_WCHUNKS):
        pltpu.sync_copy(rows_a.at[pl.ds(0, n)],
                        acc_sh.at[pl.ds(rbase + k * _CHUNK, n)])
    plsc.subcore_barrier()

    pltpu.sync_copy(ei_hbm.at[cbase], idx_a)
    pltpu.async_copy(y_hbm.at[idx_a.at[0]], rows_a, sem_a)
    pltpu.sync_copy(ei_hbm.at[cbase + 1], idx_b)

    def body(i, carry):
        j0 = 2 * i
        j1 = j0 + 1
        pltpu.async_copy(y_hbm.at[idx_b.at[0]], rows_b, sem_b)
        pltpu.make_async_copy(y_hbm.at[idx_a.at[0]], rows_a, sem_a).wait()
        pltpu.sync_copy(rows_a, acc_sh.at[idx_a.at[1]], add=True)
        jn = jnp.minimum(j0 + 2, _CHUNKS_PER_W - 1)
        pltpu.sync_copy(ei_hbm.at[cbase + jn], idx_a)
        pltpu.async_copy(y_hbm.at[idx_a.at[0]], rows_a, sem_a)
        pltpu.make_async_copy(y_hbm.at[idx_b.at[0]], rows_b, sem_b).wait()
        pltpu.sync_copy(rows_b, acc_sh.at[idx_b.at[1]], add=True)
        jm = jnp.minimum(j1 + 2, _CHUNKS_PER_W - 1)
        pltpu.sync_copy(ei_hbm.at[cbase + jm], idx_b)
        return carry

    lax.fori_loop(0, _CHUNKS_PER_W // 2, body, 0)
    pltpu.make_async_copy(y_hbm.at[idx_a.at[0]], rows_a, sem_a).wait()
    plsc.subcore_barrier()

    obase = c * _NPAD + rbase
    for k, n in enumerate(_WCHUNKS):
        pltpu.sync_copy(acc_sh.at[pl.ds(rbase + k * _CHUNK, n)],
                        rows_a.at[pl.ds(0, n)])
        pltpu.sync_copy(rows_a.at[pl.ds(0, n)],
                        acc_hbm.at[pl.ds(obase + k * _CHUNK, n)])


def _sc_cnt_body(dst_hbm, ones_hbm, zrow_hbm,
                 cnt_hbm,
                 dst_all, ones_v, cnt_sh):
    c = lax.axis_index("c")
    s = lax.axis_index("s")
    wid = c * _NS + s
    rbase = s * _RPT

    pltpu.sync_copy(dst_hbm.at[wid], dst_all)
    pltpu.sync_copy(zrow_hbm, ones_v)
    for k, n in enumerate(_WCHUNKS):
        pltpu.sync_copy(ones_v.at[pl.ds(0, n)],
                        cnt_sh.at[pl.ds(rbase + k * _CHUNK, n)])
    pltpu.sync_copy(ones_hbm, ones_v)
    plsc.subcore_barrier()

    def body(j, carry):
        pltpu.sync_copy(ones_v, cnt_sh.at[dst_all.at[j]], add=True)
        return carry

    lax.fori_loop(0, _CHUNKS_PER_W, body, 0)
    plsc.subcore_barrier()

    obase = c * _NPAD + rbase
    for k, n in enumerate(_WCHUNKS):
        pltpu.sync_copy(cnt_sh.at[pl.ds(rbase + k * _CHUNK, n)],
                        ones_v.at[pl.ds(0, n)])
        pltpu.sync_copy(ones_v.at[pl.ds(0, n)],
                        cnt_hbm.at[pl.ds(obase + k * _CHUNK, n)])


@functools.cache
def _sc_acc_call():
    return functools.partial(
        pl.kernel,
        mesh=plsc.VectorSubcoreMesh(core_axis_name="c", subcore_axis_name="s"),
        out_type=jax.ShapeDtypeStruct((_NC * _NPAD, _D), jnp.float32),
        scratch_types=[
            pltpu.VMEM((2, _CHUNK), jnp.int32),
            pltpu.VMEM((2, _CHUNK), jnp.int32),
            pltpu.VMEM((_CHUNK, _D), jnp.float32),
            pltpu.VMEM((_CHUNK, _D), jnp.float32),
            pltpu.VMEM_SHARED((_NPAD, _D), jnp.float32),
            pltpu.SemaphoreType.DMA,
            pltpu.SemaphoreType.DMA,
        ],
    )(_sc_acc_body)


@functools.cache
def _sc_cnt_call():
    return functools.partial(
        pl.kernel,
        mesh=plsc.VectorSubcoreMesh(core_axis_name="c", subcore_axis_name="s"),
        out_type=jax.ShapeDtypeStruct((_NC * _NPAD, _D), jnp.float32),
        scratch_types=[
            pltpu.VMEM((_CHUNKS_PER_W, _CHUNK), jnp.int32),
            pltpu.VMEM((_CHUNK, _D), jnp.float32),
            pltpu.VMEM_SHARED((_NPAD, _D), jnp.float32),
        ],
    )(_sc_cnt_body)


def kernel(x, edge_index, W_l, W_r, b_l):
    src = edge_index[0].astype(jnp.int32)
    dst = edge_index[1].astype(jnp.int32)
    e = src.shape[0]
    pad = _EPAD - e
    src = jnp.concatenate([src, jnp.zeros((pad,), jnp.int32)])
    dst = jnp.concatenate([dst, jnp.full((pad,), _NPAD - 1, jnp.int32)])
    ei = jnp.stack([src.reshape(-1, _CHUNK), dst.reshape(-1, _CHUNK)], axis=1)
    dst3 = dst.reshape(_NW, _CHUNKS_PER_W, _CHUNK)

    y = _matmul(x, W_l)

    ones = jnp.ones((_CHUNK, _D), jnp.float32)
    zrow = jnp.zeros((_CHUNK, _D), jnp.float32)
    acc = _sc_acc_call()(y, ei, zrow)
    cnt = _sc_cnt_call()(dst3, ones, zrow)

    a0 = acc[:_N]
    a1 = acc[_NPAD:_NPAD + _N]
    c0 = cnt[:_N]
    c1 = cnt[_NPAD:_NPAD + _N]
    return _combine(a0, a1, c0, c1, x, W_r, b_l.reshape(1, _D))
